# baseline (speedup 1.0000x reference)
import jax
import jax.numpy as jnp
from jax import lax
from jax.experimental import pallas as pl
from jax.experimental.pallas import tpu as pltpu

D = 768
H = 4
SQRT_K = 4
K = SQRT_K * SQRT_K
TOP_K = 2
NEG_INF = float("-inf")

TB = 2048


def _body(x_ref, w_ref, idx_ref, gates_ref, scores_ref):
    x = x_ref[...]
    w = w_ref[...]
    s = lax.dot_general(
        x, w, (((1,), (0,)), ((), ())), preferred_element_type=jnp.float32
    )
    st = s.T

    m1 = (st[0:4] + st[4:8] + st[8:12] + st[12:16]) * (1.0 / H)
    m2 = (st[16:20] + st[20:24] + st[24:28] + st[28:32]) * (1.0 / H)

    scores_t = jnp.concatenate(
        [m1[0:1] + m2, m1[1:2] + m2, m1[2:3] + m2, m1[3:4] + m2], axis=0
    )
    scores_ref[...] = scores_t.T

    iota = lax.broadcasted_iota(jnp.int32, (SQRT_K, TB), 0)

    v1a = jnp.max(m1, axis=0, keepdims=True)
    i1a = jnp.min(jnp.where(m1 == v1a, iota, SQRT_K), axis=0, keepdims=True)
    m1m = jnp.where(iota == i1a, NEG_INF, m1)
    v2a = jnp.max(m1m, axis=0, keepdims=True)
    i2a = jnp.min(jnp.where(m1m == v2a, iota, SQRT_K), axis=0, keepdims=True)

    v1b = jnp.max(m2, axis=0, keepdims=True)
    i1b = jnp.min(jnp.where(m2 == v1b, iota, SQRT_K), axis=0, keepdims=True)
    m2m = jnp.where(iota == i1b, NEG_INF, m2)
    v2b = jnp.max(m2m, axis=0, keepdims=True)
    i2b = jnp.min(jnp.where(m2m == v2b, iota, SQRT_K), axis=0, keepdims=True)

    top1_v = v1a + v1b
    top1_i = SQRT_K * i1a + i1b
    ca_v = v2a + v1b
    ca_i = SQRT_K * i2a + i1b
    cb_v = v1a + v2b
    cb_i = SQRT_K * i1a + i2b
    take_b = (cb_v > ca_v) | ((cb_v == ca_v) & (cb_i < ca_i))
    top2_v = jnp.where(take_b, cb_v, ca_v)
    top2_i = jnp.where(take_b, cb_i, ca_i)

    e = jnp.exp(top2_v - top1_v)
    g1 = 1.0 / (1.0 + e)
    g2 = e / (1.0 + e)

    idx_ref[...] = jnp.concatenate([top1_i, top2_i], axis=0).reshape(1, TOP_K, TB)
    gates_ref[...] = jnp.concatenate([g1, g2], axis=0).reshape(1, TOP_K, TB)


def kernel(x, W1, W2):
    n_tok = x.shape[0]
    w_cat_t = jnp.concatenate([W1, W2], axis=0).T
    grid = (n_tok // TB,)
    idx8, gates8, scores = pl.pallas_call(
        _body,
        grid=grid,
        in_specs=[
            pl.BlockSpec((TB, D), lambda i: (i, 0)),
            pl.BlockSpec((D, 2 * H * SQRT_K), lambda i: (0, 0)),
        ],
        out_specs=[
            pl.BlockSpec((1, TOP_K, TB), lambda i: (i, 0, 0)),
            pl.BlockSpec((1, TOP_K, TB), lambda i: (i, 0, 0)),
            pl.BlockSpec((TB, K), lambda i: (i, 0)),
        ],
        out_shape=[
            jax.ShapeDtypeStruct((n_tok // TB, TOP_K, TB), jnp.int32),
            jax.ShapeDtypeStruct((n_tok // TB, TOP_K, TB), jnp.float32),
            jax.ShapeDtypeStruct((n_tok, K), jnp.float32),
        ],
        compiler_params=pltpu.CompilerParams(
            dimension_semantics=("parallel",),
        ),
    )(x, w_cat_t)
    topk_idx = idx8.swapaxes(1, 2).reshape(n_tok, TOP_K)
    gates = gates8.swapaxes(1, 2).reshape(n_tok, TOP_K)
    return (topk_idx, gates, scores)


# speedup vs baseline: 1.0470x; 1.0470x over previous
import jax
import jax.numpy as jnp
from jax import lax
from jax.experimental import pallas as pl
from jax.experimental.pallas import tpu as pltpu

D = 768
H = 4
SQRT_K = 4
K = SQRT_K * SQRT_K
TOP_K = 2
NEG_INF = float("-inf")

TB = 4096


def _body(x_ref, w_ref, idx_ref, gates_ref, scores_ref):
    x = x_ref[...]
    w = w_ref[...]
    s = lax.dot_general(
        x, w, (((1,), (0,)), ((), ())), preferred_element_type=jnp.float32
    )
    st = s.T

    m1 = (st[0:4] + st[4:8] + st[8:12] + st[12:16]) * (1.0 / H)
    m2 = (st[16:20] + st[20:24] + st[24:28] + st[28:32]) * (1.0 / H)

    scores_t = jnp.concatenate(
        [m1[0:1] + m2, m1[1:2] + m2, m1[2:3] + m2, m1[3:4] + m2], axis=0
    )
    scores_ref[...] = scores_t.T

    iota = lax.broadcasted_iota(jnp.int32, (SQRT_K, TB), 0)

    v1a = jnp.max(m1, axis=0, keepdims=True)
    i1a = jnp.min(jnp.where(m1 == v1a, iota, SQRT_K), axis=0, keepdims=True)
    m1m = jnp.where(iota == i1a, NEG_INF, m1)
    v2a = jnp.max(m1m, axis=0, keepdims=True)
    i2a = jnp.min(jnp.where(m1m == v2a, iota, SQRT_K), axis=0, keepdims=True)

    v1b = jnp.max(m2, axis=0, keepdims=True)
    i1b = jnp.min(jnp.where(m2 == v1b, iota, SQRT_K), axis=0, keepdims=True)
    m2m = jnp.where(iota == i1b, NEG_INF, m2)
    v2b = jnp.max(m2m, axis=0, keepdims=True)
    i2b = jnp.min(jnp.where(m2m == v2b, iota, SQRT_K), axis=0, keepdims=True)

    top1_v = v1a + v1b
    top1_i = SQRT_K * i1a + i1b
    ca_v = v2a + v1b
    ca_i = SQRT_K * i2a + i1b
    cb_v = v1a + v2b
    cb_i = SQRT_K * i1a + i2b
    take_b = (cb_v > ca_v) | ((cb_v == ca_v) & (cb_i < ca_i))
    top2_v = jnp.where(take_b, cb_v, ca_v)
    top2_i = jnp.where(take_b, cb_i, ca_i)

    e = jnp.exp(top2_v - top1_v)
    g1 = 1.0 / (1.0 + e)
    g2 = e / (1.0 + e)

    idx_ref[...] = jnp.concatenate([top1_i, top2_i], axis=0).reshape(1, TOP_K, TB)
    gates_ref[...] = jnp.concatenate([g1, g2], axis=0).reshape(1, TOP_K, TB)


def kernel(x, W1, W2):
    n_tok = x.shape[0]
    w_cat_t = jnp.concatenate([W1, W2], axis=0).T
    grid = (n_tok // TB,)
    idx8, gates8, scores = pl.pallas_call(
        _body,
        grid=grid,
        in_specs=[
            pl.BlockSpec((TB, D), lambda i: (i, 0)),
            pl.BlockSpec((D, 2 * H * SQRT_K), lambda i: (0, 0)),
        ],
        out_specs=[
            pl.BlockSpec((1, TOP_K, TB), lambda i: (i, 0, 0)),
            pl.BlockSpec((1, TOP_K, TB), lambda i: (i, 0, 0)),
            pl.BlockSpec((TB, K), lambda i: (i, 0)),
        ],
        out_shape=[
            jax.ShapeDtypeStruct((n_tok // TB, TOP_K, TB), jnp.int32),
            jax.ShapeDtypeStruct((n_tok // TB, TOP_K, TB), jnp.float32),
            jax.ShapeDtypeStruct((n_tok, K), jnp.float32),
        ],
        compiler_params=pltpu.CompilerParams(
            dimension_semantics=("parallel",),
        ),
    )(x, w_cat_t)
    topk_idx = idx8.swapaxes(1, 2).reshape(n_tok, TOP_K)
    gates = gates8.swapaxes(1, 2).reshape(n_tok, TOP_K)
    return (topk_idx, gates, scores)
